# Initial kernel scaffold; baseline (speedup 1.0000x reference)
import jax
import jax.numpy as jnp
from jax.experimental import pallas as pl


def kernel(coordinate_grids, transforms, transforms_inv, sparse_encoding, X_data):
    raise NotImplementedError("write your pallas kernel here")


# speedup vs baseline: 7.2189x; 7.2189x over previous
import functools

import numpy as np
import jax
import jax.numpy as jnp
from jax import lax
from jax.experimental import pallas as pl
from jax.experimental.pallas import tpu as pltpu
from jax.experimental.pallas import tpu_sc as plsc

_GRID_RES = 0.08
_B = 4
_G = 64
_G3 = _G * _G * _G
_N = 100000
_NPAD = 100352
_PT = _NPAD // 16
_TOT = _B * _G3
_HTOT = 2 * _G3
_SENT = _HTOT
_OCCW = _HTOT + 16
_PACKW = _HTOT // 32
_NC, _NS = 2, 16

_mesh = plsc.VectorSubcoreMesh(
    core_axis_name="c", subcore_axis_name="s", num_cores=_NC, num_subcores=_NS
)


def _mm_body(x_ref, o_ref):
    x = x_ref[...]
    mn = jnp.min(x, axis=1)
    mx = jnp.max(x, axis=1)
    mn = jnp.broadcast_to(jnp.min(mn, axis=1, keepdims=True), (12, 128))
    mx = jnp.broadcast_to(jnp.max(mx, axis=1, keepdims=True), (12, 128))
    o_ref[0] = mn
    o_ref[1] = mx


def _minmax(cg):
    x = cg.reshape(12, 2048, 128)
    return pl.pallas_call(
        _mm_body,
        out_shape=jax.ShapeDtypeStruct((2, 12, 128), jnp.float32),
    )(x)


def _sc_body(cg_hbm, sp_hbm, par_hbm, out_hbm,
             ptx, pty, ptz, idxbuf, onesbuf, parbuf,
             stage, wordbuf, packed, qx, qy, qz, obuf,
             occ_sh, packed_sh):
    c = lax.axis_index("c")
    s = lax.axis_index("s")

    pltpu.sync_copy(par_hbm, parbuf)

    zeros16 = jnp.zeros((16,), jnp.float32)
    ones16 = jnp.ones((16,), jnp.float32)

    def _zf(i, _):
        stage[pl.ds(i * 16, 16)] = zeros16
        return 0
    lax.fori_loop(0, 256, _zf, 0)

    def _of(i, _):
        onesbuf[pl.ds(i * 16, 16)] = ones16
        return 0
    lax.fori_loop(0, 8, _of, 0)

    def _zc(i, _):
        pltpu.sync_copy(stage, occ_sh.at[pl.ds(s * 32768 + i * 4096, 4096)])
        return 0
    lax.fori_loop(0, 8, _zc, 0)
    plsc.subcore_barrier()

    for bl in range(2):
        b = c * 2 + bl
        pb = b * 192
        mvx = parbuf[pl.ds(pb + 0, 16)]
        mvy = parbuf[pl.ds(pb + 16, 16)]
        mvz = parbuf[pl.ds(pb + 32, 16)]
        svx = parbuf[pl.ds(pb + 48, 16)]
        svy = parbuf[pl.ds(pb + 64, 16)]
        svz = parbuf[pl.ds(pb + 80, 16)]
        off = s * _PT
        pltpu.sync_copy(sp_hbm.at[pl.ds((0 * _B + b) * _NPAD + off, _PT)], ptx)
        pltpu.sync_copy(sp_hbm.at[pl.ds((1 * _B + b) * _NPAD + off, _PT)], pty)
        pltpu.sync_copy(sp_hbm.at[pl.ds((2 * _B + b) * _NPAD + off, _PT)], ptz)
        bbase = bl * _G3

        def _sc_chunk(j, _, mvx=mvx, mvy=mvy, mvz=mvz,
                      svx=svx, svy=svy, svz=svz, bbase=bbase):
            for k in range(8):
                o = j * 128 + k * 16
                x = ptx[pl.ds(o, 16)] - mvx
                y = pty[pl.ds(o, 16)] - mvy
                z = ptz[pl.ds(o, 16)] - mvz
                valid = ((x >= 0.0) & (x < svx) & (y >= 0.0) & (y < svy)
                         & (z >= 0.0) & (z < svz))
                xi = x.astype(jnp.int32)
                yi = y.astype(jnp.int32)
                zi = z.astype(jnp.int32)
                idx = bbase + (xi << 12) + (yi << 6) + zi
                idxbuf[pl.ds(k * 16, 16)] = jnp.where(valid, idx, _SENT)
            pltpu.sync_copy(onesbuf, occ_sh.at[idxbuf], add=True)
            return 0
        lax.fori_loop(0, _PT // 128, _sc_chunk, 0)
    plsc.subcore_barrier()

    iota32 = lax.iota(jnp.int32, 16) * 32

    def _pk_chunk(i, _):
        pltpu.sync_copy(occ_sh.at[pl.ds(s * 32768 + i * 4096, 4096)], stage)
        for g in range(8):
            w = jnp.zeros((16,), jnp.int32)
            for j in range(32):
                vals = plsc.load_gather(stage, [iota32 + (g * 512 + j)])
                bitj = int(np.int32(np.uint32(1) << j))
                w = w | jnp.where(vals > 0.0, bitj, 0)
            wordbuf[pl.ds(g * 16, 16)] = w
        pltpu.sync_copy(wordbuf, packed_sh.at[pl.ds(s * 1024 + i * 128, 128)])
        return 0
    lax.fori_loop(0, 8, _pk_chunk, 0)
    plsc.subcore_barrier()
    pltpu.sync_copy(packed_sh, packed)

    bl = s // 8
    b = c * 2 + bl
    sub = s % 8
    pb = b * 192
    sgx = parbuf[pl.ds(pb + 48, 16)]
    sgy = parbuf[pl.ds(pb + 64, 16)]
    sgz = parbuf[pl.ds(pb + 80, 16)]
    bx = parbuf[pl.ds(pb + 96, 16)]
    by = parbuf[pl.ds(pb + 112, 16)]
    bz = parbuf[pl.ds(pb + 128, 16)]
    cx = parbuf[pl.ds(pb + 144, 16)]
    cy = parbuf[pl.ds(pb + 160, 16)]
    cz = parbuf[pl.ds(pb + 176, 16)]
    smx = sgx.astype(jnp.int32) - 1
    smy = sgy.astype(jnp.int32) - 1
    smz = sgz.astype(jnp.int32) - 1
    bbase = b * _G3
    lbase = bl * _G3
    qoff = sub * 32768
    zero16i = jnp.zeros((16,), jnp.int32)

    def _g_chunk(i, _):
        qb = qoff + i * 2048
        pltpu.sync_copy(cg_hbm.at[pl.ds((b * 3 + 0) * _G3 + qb, 2048)], qx)
        pltpu.sync_copy(cg_hbm.at[pl.ds((b * 3 + 1) * _G3 + qb, 2048)], qy)
        pltpu.sync_copy(cg_hbm.at[pl.ds((b * 3 + 2) * _G3 + qb, 2048)], qz)

        def _g_vec(k, _):
            o = k * 16
            vx = (qx[pl.ds(o, 16)] - bx) * cx
            vy = (qy[pl.ds(o, 16)] - by) * cy
            vz = (qz[pl.ds(o, 16)] - bz) * cz
            ivx = vx.astype(jnp.int32)
            ivy = vy.astype(jnp.int32)
            ivz = vz.astype(jnp.int32)
            ivx = jnp.where(ivx.astype(jnp.float32) > vx, ivx - 1, ivx)
            ivy = jnp.where(ivy.astype(jnp.float32) > vy, ivy - 1, ivy)
            ivz = jnp.where(ivz.astype(jnp.float32) > vz, ivz - 1, ivz)
            fx = vx - ivx.astype(jnp.float32)
            fy = vy - ivy.astype(jnp.float32)
            fz = vz - ivz.astype(jnp.float32)
            ix0 = jnp.minimum(jnp.maximum(ivx, zero16i), smx)
            ix1 = jnp.minimum(jnp.maximum(ivx + 1, zero16i), smx)
            iy0 = jnp.minimum(jnp.maximum(ivy, zero16i), smy)
            iy1 = jnp.minimum(jnp.maximum(ivy + 1, zero16i), smy)
            iz0 = jnp.minimum(jnp.maximum(ivz, zero16i), smz)
            iz1 = jnp.minimum(jnp.maximum(ivz + 1, zero16i), smz)
            a00 = lbase + (ix0 << 12) + (iy0 << 6)
            a01 = lbase + (ix0 << 12) + (iy1 << 6)
            a10 = lbase + (ix1 << 12) + (iy0 << 6)
            a11 = lbase + (ix1 << 12) + (iy1 << 6)
            wx0 = 1.0 - fx
            wy0 = 1.0 - fy
            wz0 = 1.0 - fz
            acc = jnp.zeros((16,), jnp.float32)
            for aij, wxy in ((a00, wx0 * wy0), (a01, wx0 * fy),
                             (a10, fx * wy0), (a11, fx * fy)):
                for izc, wz in ((iz0, wz0), (iz1, fz)):
                    f = aij + izc
                    wvec = plsc.load_gather(packed, [f >> 5])
                    bit = (wvec >> (f & 31)) & 1
                    acc = acc + (wxy * wz) * bit.astype(jnp.float32)
            obuf[pl.ds(o, 16)] = jnp.where(acc > 0.5, 1, 0)
            return 0
        lax.fori_loop(0, 128, _g_vec, 0)
        pltpu.sync_copy(obuf, out_hbm.at[pl.ds(bbase + qb, 2048)])
        return 0
    lax.fori_loop(0, 16, _g_chunk, 0)


_sc_call = functools.partial(
    pl.kernel,
    out_type=jax.ShapeDtypeStruct((_TOT,), jnp.int32),
    mesh=_mesh,
    compiler_params=pltpu.CompilerParams(needs_layout_passes=False),
    scratch_types=[
        pltpu.VMEM((_PT,), jnp.float32),
        pltpu.VMEM((_PT,), jnp.float32),
        pltpu.VMEM((_PT,), jnp.float32),
        pltpu.VMEM((128,), jnp.int32),
        pltpu.VMEM((128,), jnp.float32),
        pltpu.VMEM((768,), jnp.float32),
        pltpu.VMEM((4096,), jnp.float32),
        pltpu.VMEM((128,), jnp.int32),
        pltpu.VMEM((_PACKW,), jnp.int32),
        pltpu.VMEM((2048,), jnp.float32),
        pltpu.VMEM((2048,), jnp.float32),
        pltpu.VMEM((2048,), jnp.float32),
        pltpu.VMEM((2048,), jnp.int32),
        pltpu.VMEM_SHARED((_OCCW,), jnp.float32),
        pltpu.VMEM_SHARED((_PACKW,), jnp.int32),
    ],
)(_sc_body)


def kernel(coordinate_grids, transforms, transforms_inv, sparse_encoding, X_data):
    mm = _minmax(coordinate_grids)
    mn = mm[0, :, 0].reshape(_B, 3)
    mx = mm[1, :, 0].reshape(_B, 3)

    def _matvec_f32(M, h):
        return (((M[:, :, 0] * h[:, 0:1] + M[:, :, 1] * h[:, 1:2])
                 + M[:, :, 2] * h[:, 2:3]) + M[:, :, 3] * h[:, 3:4])

    def _bf(x):
        return x.astype(jnp.bfloat16).astype(jnp.float32)

    max_size_grid = (mx + _GRID_RES - mn).max(axis=0)
    min_homo = jnp.concatenate([mn, jnp.ones((_B, 1), jnp.float32)], axis=1)
    min_voxel_idx = jnp.floor(
        jnp.einsum('bij,bj->bi', transforms_inv, min_homo))[:, :3]
    min_voxel_idx = jnp.where(min_voxel_idx < 0, 0.0, min_voxel_idx)
    svg = jnp.ceil(
        jnp.einsum('bij,j->bi', transforms_inv[:, :3, :3], max_size_grid
                   ).max(axis=0))
    min_idx_homo = jnp.concatenate(
        [min_voxel_idx, jnp.ones((_B, 1), jnp.float32)], axis=1)
    base = _matvec_f32(transforms[:, :3, :], min_idx_homo)
    extent = jnp.einsum('bij,j->bi', transforms[:, :3, :3], svg)
    cell = extent / svg
    inv_cell = 1.0 / cell

    svg_b = jnp.broadcast_to(svg[None, :], (_B, 3))
    pvals = jnp.concatenate([min_voxel_idx, svg_b, base, inv_cell], axis=1)
    params = jnp.broadcast_to(pvals[:, :, None], (_B, 12, 16)).reshape(-1)

    cg_flat = coordinate_grids.reshape(-1)
    sp = jnp.transpose(sparse_encoding, (2, 0, 1))
    sp = jnp.pad(sp, ((0, 0), (0, 0), (0, _NPAD - _N)),
                 constant_values=-1.0).reshape(-1)

    y = _sc_call(cg_flat, sp, params)
    Y = y.reshape(_B, _G, _G, _G) != 0
    return X_data, Y, coordinate_grids
